# baseline (speedup 1.0000x reference)
import jax
import jax.numpy as jnp
from jax.experimental import pallas as pl
from jax.experimental.pallas import tpu as pltpu

_NUM_NEIGHBORS = 9
_D = 128
_KB1 = 2048
_KB2 = 4096


def _stage1_kernel(x_ref, y_ref, min_ref, idx_ref, *, k_total, n_blocks):
    k = pl.program_id(0)
    x = x_ref[...]
    y = y_ref[...]
    xn = jnp.sum(x * x, axis=1, keepdims=True)
    yn = jnp.sum(y * y, axis=1)
    prod = jax.lax.dot_general(
        x, y, (((1,), (1,)), ((), ())),
        preferred_element_type=jnp.float32,
    )
    res = xn - 2.0 * prod + yn[None, :]
    dist = jnp.sqrt(jnp.maximum(res, 0.0))
    col = jax.lax.broadcasted_iota(jnp.int32, dist.shape, 1) + k * _KB1
    dist = jnp.where(col < k_total, dist, jnp.inf)
    bmin = jnp.min(dist, axis=1)
    bidx = jnp.min(
        jnp.where(dist == bmin[:, None], col, jnp.int32(k_total)), axis=1
    )

    @pl.when(k == 0)
    def _():
        min_ref[...] = bmin[None, :]
        idx_ref[...] = bidx[None, :]

    @pl.when(k > 0)
    def _():
        cur = min_ref[0, :]
        better = bmin < cur
        min_ref[...] = jnp.where(better, bmin, cur)[None, :]
        idx_ref[...] = jnp.where(better, bidx, idx_ref[0, :])[None, :]


def _stage2_kernel(s_ref, emb_ref, nn_ref, y_ref, ps_ref, pred_ref,
                   d2_scr, dr_scr, *, k_total, n_blocks):
    k = pl.program_id(0)
    y = y_ref[...]
    yn = jnp.sum(y * y, axis=1)
    q = jnp.concatenate([nn_ref[0], emb_ref[0]], axis=0)
    qn = jnp.sum(q * q, axis=1)
    prod = jax.lax.dot_general(
        y, q, (((1,), (1,)), ((), ())),
        preferred_element_type=jnp.float32,
    )
    res = qn[None, :] - 2.0 * prod + yn[:, None]
    dist = jnp.sqrt(jnp.maximum(res, 0.0))
    col = jax.lax.broadcasted_iota(jnp.int32, (_KB2,), 0) + k * _KB2
    d2_scr[k, :] = jnp.where(col < k_total, dist[:, 0], jnp.inf)
    dr_scr[k, :] = dist[:, 1]

    @pl.when(k == n_blocks - 1)
    def _():
        d2 = d2_scr[...]
        dr = dr_scr[...]
        fi = (jax.lax.broadcasted_iota(jnp.int32, d2.shape, 0) * _KB2
              + jax.lax.broadcasted_iota(jnp.int32, d2.shape, 1))
        big = jnp.int32(2 ** 30)
        d3 = []
        for _ in range(_NUM_NEIGHBORS):
            m = jnp.min(d2)
            fmin = jnp.min(jnp.where(d2 == m, fi, big))
            sel = fi == fmin
            d3.append(jnp.sum(jnp.where(sel, dr, 0.0)))
            d2 = jnp.where(sel, jnp.inf, d2)
        mx = d3[0]
        for v in d3[1:]:
            mx = jnp.maximum(mx, v)
        denom = jnp.float32(0.0)
        for v in d3:
            denom = denom + jnp.exp(v - mx)
        w = 1.0 - jnp.exp(d3[0] - mx) / denom
        score = jnp.max(ps_ref[...])
        pred_ref[...] = jnp.reshape(w * score, (1, 1))


def kernel(embedding, memory_bank):
    n, d = embedding.shape
    k_total = memory_bank.shape[0]
    g1 = pl.cdiv(k_total, _KB1)
    g2 = pl.cdiv(k_total, _KB2)

    import functools
    ps, loc = pl.pallas_call(
        functools.partial(_stage1_kernel, k_total=k_total, n_blocks=g1),
        grid=(g1,),
        in_specs=[
            pl.BlockSpec((n, d), lambda k: (0, 0)),
            pl.BlockSpec((_KB1, d), lambda k: (k, 0)),
        ],
        out_specs=[
            pl.BlockSpec((1, n), lambda k: (0, 0)),
            pl.BlockSpec((1, n), lambda k: (0, 0)),
        ],
        out_shape=[
            jax.ShapeDtypeStruct((1, n), jnp.float32),
            jax.ShapeDtypeStruct((1, n), jnp.int32),
        ],
    )(embedding, memory_bank)

    mp = jnp.argmax(ps[0]).astype(jnp.int32)
    nn_index = loc[0, mp]
    scalars = jnp.stack([mp, nn_index])

    pred = pl.pallas_call(
        functools.partial(_stage2_kernel, k_total=k_total, n_blocks=g2),
        grid_spec=pltpu.PrefetchScalarGridSpec(
            num_scalar_prefetch=1,
            grid=(g2,),
            in_specs=[
                pl.BlockSpec((1, 1, d), lambda k, s: (s[0], 0, 0)),
                pl.BlockSpec((1, 1, d), lambda k, s: (s[1], 0, 0)),
                pl.BlockSpec((_KB2, d), lambda k, s: (k, 0)),
                pl.BlockSpec((1, n), lambda k, s: (0, 0)),
            ],
            out_specs=pl.BlockSpec((1, 1), lambda k, s: (0, 0)),
            scratch_shapes=[
                pltpu.VMEM((g2, _KB2), jnp.float32),
                pltpu.VMEM((g2, _KB2), jnp.float32),
            ],
        ),
        out_shape=jax.ShapeDtypeStruct((1, 1), jnp.float32),
    )(scalars, embedding.reshape(n, 1, d), memory_bank.reshape(k_total, 1, d),
      memory_bank, ps)

    pred_score = pred.reshape(1)
    anomaly_map = ps.reshape(1, 1, 28, 28)
    return pred_score, anomaly_map, loc


# speedup vs baseline: 2.2466x; 2.2466x over previous
import functools

import jax
import jax.numpy as jnp
from jax.experimental import pallas as pl
from jax.experimental.pallas import tpu as pltpu

_NUM_NEIGHBORS = 9
_KB1 = 2000
_KB2 = 8000
_BIG = 2 ** 30


def _stage1_kernel(xm2_ref, xn_ref, y_ref, yn_ref, iota_ref,
                   s_ref, idx_ref, sc_ref, *, n_blocks):
    k = pl.program_id(0)
    prod = jax.lax.dot_general(
        xm2_ref[...], y_ref[...], (((1,), (1,)), ((), ())),
        preferred_element_type=jnp.float32,
    )
    res = (xn_ref[...] + prod) + yn_ref[0]
    bmin = jnp.min(res, axis=1)
    base = jnp.maximum(bmin, 0.0)
    s = jnp.sqrt(base)
    bb = jax.lax.bitcast_convert_type(base, jnp.int32)
    hi = base
    for d in range(1, 7):
        hc = jax.lax.bitcast_convert_type(bb + d, jnp.float32)
        hi = jnp.where(jnp.sqrt(hc) == s, hc, hi)
    ii = jnp.where(res <= hi[:, None], iota_ref[0], _BIG)
    jloc = jnp.min(ii, axis=1) + k * _KB1

    @pl.when(k == 0)
    def _():
        s_ref[...] = s[None, :]
        idx_ref[...] = jloc[None, :]

    @pl.when(k > 0)
    def _():
        cur = s_ref[0, :]
        better = s < cur
        s_ref[...] = jnp.where(better, s, cur)[None, :]
        idx_ref[...] = jnp.where(better, jloc, idx_ref[0, :])[None, :]

    @pl.when(k == n_blocks - 1)
    def _():
        ps = s_ref[0, :]
        loc = idx_ref[0, :]
        i_n = jax.lax.broadcasted_iota(jnp.int32, ps.shape, 0)
        mp = jnp.min(jnp.where(ps == jnp.max(ps), i_n, _BIG))
        nn = jnp.sum(jnp.where(i_n == mp, loc, 0))
        sc_ref[...] = jnp.stack([mp, nn])[None, :]


def _stage2_kernel(s_ref, emb_ref, nnrow_ref, y_ref, yn_ref, ps_ref,
                   pred_ref, d2_scr, dr_scr, *, n_blocks):
    k = pl.program_id(0)
    q = jnp.concatenate([nnrow_ref[0], emb_ref[0]], axis=0)
    qn = jnp.sum(q * q, axis=1, keepdims=True)
    prod = jax.lax.dot_general(
        -2.0 * q, y_ref[...], (((1,), (1,)), ((), ())),
        preferred_element_type=jnp.float32,
    )
    res = (qn + prod) + yn_ref[0]
    dist = jnp.sqrt(jnp.maximum(res, 0.0))
    d2_scr[k, :] = dist[0, :]
    dr_scr[k, :] = dist[1, :]

    @pl.when(k == n_blocks - 1)
    def _():
        d2 = d2_scr[...]
        dr = dr_scr[...]
        fi = (jax.lax.broadcasted_iota(jnp.int32, d2.shape, 0) * _KB2
              + jax.lax.broadcasted_iota(jnp.int32, d2.shape, 1))
        d3 = []
        for _ in range(_NUM_NEIGHBORS):
            m = jnp.min(d2)
            fmin = jnp.min(jnp.where(d2 == m, fi, _BIG))
            sel = fi == fmin
            d3.append(jnp.sum(jnp.where(sel, dr, 0.0)))
            d2 = jnp.where(sel, jnp.inf, d2)
        mx = d3[0]
        for v in d3[1:]:
            mx = jnp.maximum(mx, v)
        denom = jnp.float32(0.0)
        for v in d3:
            denom = denom + jnp.exp(v - mx)
        w = 1.0 - jnp.exp(d3[0] - mx) / denom
        score = jnp.max(ps_ref[...])
        pred_ref[...] = jnp.reshape(w * score, (1, 1))


def kernel(embedding, memory_bank):
    n, d = embedding.shape
    k_total = memory_bank.shape[0]
    g1 = k_total // _KB1
    g2 = k_total // _KB2

    xn = jnp.sum(embedding ** 2, axis=-1, keepdims=True)
    yn = jnp.sum(memory_bank ** 2, axis=-1)
    xm2 = -2.0 * embedding
    yn1 = yn.reshape(g1, 1, _KB1)
    yn2 = yn.reshape(g2, 1, _KB2)
    iota1 = jax.lax.iota(jnp.int32, _KB1).reshape(1, 1, _KB1)

    ps, loc, sc = pl.pallas_call(
        functools.partial(_stage1_kernel, n_blocks=g1),
        grid=(g1,),
        in_specs=[
            pl.BlockSpec((n, d), lambda k: (0, 0)),
            pl.BlockSpec((n, 1), lambda k: (0, 0)),
            pl.BlockSpec((_KB1, d), lambda k: (k, 0)),
            pl.BlockSpec((1, 1, _KB1), lambda k: (k, 0, 0)),
            pl.BlockSpec((1, 1, _KB1), lambda k: (0, 0, 0)),
        ],
        out_specs=[
            pl.BlockSpec((1, n), lambda k: (0, 0)),
            pl.BlockSpec((1, n), lambda k: (0, 0)),
            pl.BlockSpec((1, 2), lambda k: (0, 0)),
        ],
        out_shape=[
            jax.ShapeDtypeStruct((1, n), jnp.float32),
            jax.ShapeDtypeStruct((1, n), jnp.int32),
            jax.ShapeDtypeStruct((1, 2), jnp.int32),
        ],
    )(xm2, xn, memory_bank, yn1, iota1)

    pred = pl.pallas_call(
        functools.partial(_stage2_kernel, n_blocks=g2),
        grid_spec=pltpu.PrefetchScalarGridSpec(
            num_scalar_prefetch=1,
            grid=(g2,),
            in_specs=[
                pl.BlockSpec((1, 1, d), lambda k, s: (s[0], 0, 0)),
                pl.BlockSpec((1, 1, d), lambda k, s: (s[1], 0, 0)),
                pl.BlockSpec((_KB2, d), lambda k, s: (k, 0)),
                pl.BlockSpec((1, 1, _KB2), lambda k, s: (k, 0, 0)),
                pl.BlockSpec((1, n), lambda k, s: (0, 0)),
            ],
            out_specs=pl.BlockSpec((1, 1), lambda k, s: (0, 0)),
            scratch_shapes=[
                pltpu.VMEM((g2, _KB2), jnp.float32),
                pltpu.VMEM((g2, _KB2), jnp.float32),
            ],
        ),
        out_shape=jax.ShapeDtypeStruct((1, 1), jnp.float32),
    )(sc.reshape(2), embedding.reshape(n, 1, d),
      memory_bank.reshape(k_total, 1, d), memory_bank, yn2, ps)

    pred_score = pred.reshape(1)
    anomaly_map = ps.reshape(1, 1, 28, 28)
    return pred_score, anomaly_map, loc


# speedup vs baseline: 2.6325x; 1.1718x over previous
import functools

import jax
import jax.numpy as jnp
from jax.experimental import pallas as pl
from jax.experimental.pallas import tpu as pltpu

_NUM_NEIGHBORS = 9
_KB1 = 2000
_KB2 = 8000
_BIG = 2 ** 30


def _stage1_kernel(xm2_ref, xn_ref, y_ref, yn_ref, iota_ref,
                   s_ref, idx_ref, sc_ref, *, n_blocks):
    k = pl.program_id(0)
    prod = jax.lax.dot_general(
        y_ref[...], xm2_ref[...], (((1,), (1,)), ((), ())),
        preferred_element_type=jnp.float32,
    )
    res = (xn_ref[...] + prod) + yn_ref[...]
    bmin = jnp.min(res, axis=0)
    base = jnp.maximum(bmin, 0.0)
    s = jnp.sqrt(base)
    bb = jax.lax.bitcast_convert_type(base, jnp.int32)
    hi = base
    for d in range(1, 7):
        hc = jax.lax.bitcast_convert_type(bb + d, jnp.float32)
        hi = jnp.where(jnp.sqrt(hc) == s, hc, hi)
    ii = jnp.where(res <= hi[None, :], iota_ref[...], _BIG)
    jloc = jnp.min(ii, axis=0) + k * _KB1

    @pl.when(k == 0)
    def _():
        s_ref[...] = s[None, :]
        idx_ref[...] = jloc[None, :]

    @pl.when(k > 0)
    def _():
        cur = s_ref[0, :]
        better = s < cur
        s_ref[...] = jnp.where(better, s, cur)[None, :]
        idx_ref[...] = jnp.where(better, jloc, idx_ref[0, :])[None, :]

    @pl.when(k == n_blocks - 1)
    def _():
        ps = s_ref[0, :]
        loc = idx_ref[0, :]
        i_n = jax.lax.broadcasted_iota(jnp.int32, ps.shape, 0)
        mp = jnp.min(jnp.where(ps == jnp.max(ps), i_n, _BIG))
        nn = jnp.sum(jnp.where(i_n == mp, loc, 0))
        sc_ref[...] = jnp.stack([mp, nn])[None, :]


def _stage2_kernel(s_ref, emb_ref, nnrow_ref, y_ref, yn_ref, ps_ref,
                   pred_ref, d2_scr, dr_scr, *, n_blocks):
    k = pl.program_id(0)
    q = jnp.concatenate([nnrow_ref[0], emb_ref[0]], axis=0)
    qn = jnp.sum(q * q, axis=1, keepdims=True)
    prod = jax.lax.dot_general(
        -2.0 * q, y_ref[...], (((1,), (1,)), ((), ())),
        preferred_element_type=jnp.float32,
    )
    res = (qn + prod) + yn_ref[0]
    dist = jnp.sqrt(jnp.maximum(res, 0.0))
    d2_scr[k, :] = dist[0, :]
    dr_scr[k, :] = dist[1, :]

    @pl.when(k == n_blocks - 1)
    def _():
        d2 = d2_scr[...]
        dr = dr_scr[...]
        fi = (jax.lax.broadcasted_iota(jnp.int32, d2.shape, 0) * _KB2
              + jax.lax.broadcasted_iota(jnp.int32, d2.shape, 1))
        d3 = []
        for _ in range(_NUM_NEIGHBORS):
            m = jnp.min(d2)
            fmin = jnp.min(jnp.where(d2 == m, fi, _BIG))
            sel = fi == fmin
            d3.append(jnp.sum(jnp.where(sel, dr, 0.0)))
            d2 = jnp.where(sel, jnp.inf, d2)
        mx = d3[0]
        for v in d3[1:]:
            mx = jnp.maximum(mx, v)
        denom = jnp.float32(0.0)
        for v in d3:
            denom = denom + jnp.exp(v - mx)
        w = 1.0 - jnp.exp(d3[0] - mx) / denom
        score = jnp.max(ps_ref[...])
        pred_ref[...] = jnp.reshape(w * score, (1, 1))


def kernel(embedding, memory_bank):
    n, d = embedding.shape
    k_total = memory_bank.shape[0]
    g1 = k_total // _KB1
    g2 = k_total // _KB2

    xn = jnp.sum(embedding ** 2, axis=-1, keepdims=True)
    yn = jnp.sum(memory_bank ** 2, axis=-1)
    xm2 = -2.0 * embedding
    xn_row = xn.reshape(1, n)
    yn_col = yn.reshape(k_total, 1)
    yn2 = yn.reshape(g2, 1, _KB2)
    iota1 = jax.lax.iota(jnp.int32, _KB1).reshape(_KB1, 1)

    ps, loc, sc = pl.pallas_call(
        functools.partial(_stage1_kernel, n_blocks=g1),
        grid=(g1,),
        in_specs=[
            pl.BlockSpec((n, d), lambda k: (0, 0)),
            pl.BlockSpec((1, n), lambda k: (0, 0)),
            pl.BlockSpec((_KB1, d), lambda k: (k, 0)),
            pl.BlockSpec((_KB1, 1), lambda k: (k, 0)),
            pl.BlockSpec((_KB1, 1), lambda k: (0, 0)),
        ],
        out_specs=[
            pl.BlockSpec((1, n), lambda k: (0, 0)),
            pl.BlockSpec((1, n), lambda k: (0, 0)),
            pl.BlockSpec((1, 2), lambda k: (0, 0)),
        ],
        out_shape=[
            jax.ShapeDtypeStruct((1, n), jnp.float32),
            jax.ShapeDtypeStruct((1, n), jnp.int32),
            jax.ShapeDtypeStruct((1, 2), jnp.int32),
        ],
    )(xm2, xn_row, memory_bank, yn_col, iota1)

    pred = pl.pallas_call(
        functools.partial(_stage2_kernel, n_blocks=g2),
        grid_spec=pltpu.PrefetchScalarGridSpec(
            num_scalar_prefetch=1,
            grid=(g2,),
            in_specs=[
                pl.BlockSpec((1, 1, d), lambda k, s: (s[0], 0, 0)),
                pl.BlockSpec((1, 1, d), lambda k, s: (s[1], 0, 0)),
                pl.BlockSpec((_KB2, d), lambda k, s: (k, 0)),
                pl.BlockSpec((1, 1, _KB2), lambda k, s: (k, 0, 0)),
                pl.BlockSpec((1, n), lambda k, s: (0, 0)),
            ],
            out_specs=pl.BlockSpec((1, 1), lambda k, s: (0, 0)),
            scratch_shapes=[
                pltpu.VMEM((g2, _KB2), jnp.float32),
                pltpu.VMEM((g2, _KB2), jnp.float32),
            ],
        ),
        out_shape=jax.ShapeDtypeStruct((1, 1), jnp.float32),
    )(sc.reshape(2), embedding.reshape(n, 1, d),
      memory_bank.reshape(k_total, 1, d), memory_bank, yn2, ps)

    pred_score = pred.reshape(1)
    anomaly_map = ps.reshape(1, 1, 28, 28)
    return pred_score, anomaly_map, loc


# speedup vs baseline: 2.7377x; 1.0400x over previous
import functools

import jax
import jax.numpy as jnp
from jax.experimental import pallas as pl
from jax.experimental.pallas import tpu as pltpu

_NUM_NEIGHBORS = 9
_KB1 = 4000
_KB2 = 20000
_BIG = 2 ** 30


def _stage1_kernel(xm2_ref, xn_ref, y_ref, yn_ref, iota_ref,
                   s_ref, idx_ref, sc_ref, *, n_blocks):
    k = pl.program_id(0)
    prod = jax.lax.dot_general(
        y_ref[...], xm2_ref[...], (((1,), (1,)), ((), ())),
        preferred_element_type=jnp.float32,
    )
    res = (xn_ref[...] + prod) + yn_ref[...]
    bmin = jnp.min(res, axis=0)
    base = jnp.maximum(bmin, 0.0)
    s = jnp.sqrt(base)
    bb = jax.lax.bitcast_convert_type(base, jnp.int32)
    hi = base
    for d in range(1, 7):
        hc = jax.lax.bitcast_convert_type(bb + d, jnp.float32)
        hi = jnp.where(jnp.sqrt(hc) == s, hc, hi)
    ii = jnp.where(res <= hi[None, :], iota_ref[...], _BIG)
    jloc = jnp.min(ii, axis=0) + k * _KB1

    @pl.when(k == 0)
    def _():
        s_ref[...] = s[None, :]
        idx_ref[...] = jloc[None, :]

    @pl.when(k > 0)
    def _():
        cur = s_ref[0, :]
        better = s < cur
        s_ref[...] = jnp.where(better, s, cur)[None, :]
        idx_ref[...] = jnp.where(better, jloc, idx_ref[0, :])[None, :]

    @pl.when(k == n_blocks - 1)
    def _():
        ps = s_ref[0, :]
        loc = idx_ref[0, :]
        i_n = jax.lax.broadcasted_iota(jnp.int32, ps.shape, 0)
        mp = jnp.min(jnp.where(ps == jnp.max(ps), i_n, _BIG))
        nn = jnp.sum(jnp.where(i_n == mp, loc, 0))
        sc_ref[...] = jnp.stack([mp, nn])[None, :]


def _stage2_kernel(s_ref, emb_ref, nnrow_ref, y_ref, yn_ref, ps_ref,
                   pred_ref, d2_scr, dr_scr, *, n_blocks):
    k = pl.program_id(0)
    q = jnp.concatenate([nnrow_ref[0], emb_ref[0]], axis=0)
    qn = jnp.sum(q * q, axis=1, keepdims=True)
    prod = jax.lax.dot_general(
        -2.0 * q, y_ref[...], (((1,), (1,)), ((), ())),
        preferred_element_type=jnp.float32,
    )
    res = (qn + prod) + yn_ref[0]
    dist = jnp.sqrt(jnp.maximum(res, 0.0))
    d2_scr[k, :] = dist[0, :]
    dr_scr[k, :] = dist[1, :]

    @pl.when(k == n_blocks - 1)
    def _():
        d2 = d2_scr[...]
        dr = dr_scr[...]
        fi = (jax.lax.broadcasted_iota(jnp.int32, d2.shape, 0) * _KB2
              + jax.lax.broadcasted_iota(jnp.int32, d2.shape, 1))
        d3 = []
        for _ in range(_NUM_NEIGHBORS):
            m = jnp.min(d2)
            fmin = jnp.min(jnp.where(d2 == m, fi, _BIG))
            sel = fi == fmin
            d3.append(jnp.sum(jnp.where(sel, dr, 0.0)))
            d2 = jnp.where(sel, jnp.inf, d2)
        mx = d3[0]
        for v in d3[1:]:
            mx = jnp.maximum(mx, v)
        denom = jnp.float32(0.0)
        for v in d3:
            denom = denom + jnp.exp(v - mx)
        w = 1.0 - jnp.exp(d3[0] - mx) / denom
        score = jnp.max(ps_ref[...])
        pred_ref[...] = jnp.reshape(w * score, (1, 1))


def kernel(embedding, memory_bank):
    n, d = embedding.shape
    k_total = memory_bank.shape[0]
    g1 = k_total // _KB1
    g2 = k_total // _KB2

    xn = jnp.sum(embedding ** 2, axis=-1, keepdims=True)
    yn = jnp.sum(memory_bank ** 2, axis=-1)
    xm2 = -2.0 * embedding
    xn_row = xn.reshape(1, n)
    yn_col = yn.reshape(k_total, 1)
    yn2 = yn.reshape(g2, 1, _KB2)
    iota1 = jax.lax.iota(jnp.int32, _KB1).reshape(_KB1, 1)

    ps, loc, sc = pl.pallas_call(
        functools.partial(_stage1_kernel, n_blocks=g1),
        grid=(g1,),
        in_specs=[
            pl.BlockSpec((n, d), lambda k: (0, 0)),
            pl.BlockSpec((1, n), lambda k: (0, 0)),
            pl.BlockSpec((_KB1, d), lambda k: (k, 0)),
            pl.BlockSpec((_KB1, 1), lambda k: (k, 0)),
            pl.BlockSpec((_KB1, 1), lambda k: (0, 0)),
        ],
        out_specs=[
            pl.BlockSpec((1, n), lambda k: (0, 0)),
            pl.BlockSpec((1, n), lambda k: (0, 0)),
            pl.BlockSpec((1, 2), lambda k: (0, 0)),
        ],
        out_shape=[
            jax.ShapeDtypeStruct((1, n), jnp.float32),
            jax.ShapeDtypeStruct((1, n), jnp.int32),
            jax.ShapeDtypeStruct((1, 2), jnp.int32),
        ],
    )(xm2, xn_row, memory_bank, yn_col, iota1)

    pred = pl.pallas_call(
        functools.partial(_stage2_kernel, n_blocks=g2),
        grid_spec=pltpu.PrefetchScalarGridSpec(
            num_scalar_prefetch=1,
            grid=(g2,),
            in_specs=[
                pl.BlockSpec((1, 1, d), lambda k, s: (s[0], 0, 0)),
                pl.BlockSpec((1, 1, d), lambda k, s: (s[1], 0, 0)),
                pl.BlockSpec((_KB2, d), lambda k, s: (k, 0)),
                pl.BlockSpec((1, 1, _KB2), lambda k, s: (k, 0, 0)),
                pl.BlockSpec((1, n), lambda k, s: (0, 0)),
            ],
            out_specs=pl.BlockSpec((1, 1), lambda k, s: (0, 0)),
            scratch_shapes=[
                pltpu.VMEM((g2, _KB2), jnp.float32),
                pltpu.VMEM((g2, _KB2), jnp.float32),
            ],
        ),
        out_shape=jax.ShapeDtypeStruct((1, 1), jnp.float32),
    )(sc.reshape(2), embedding.reshape(n, 1, d),
      memory_bank.reshape(k_total, 1, d), memory_bank, yn2, ps)

    pred_score = pred.reshape(1)
    anomaly_map = ps.reshape(1, 1, 28, 28)
    return pred_score, anomaly_map, loc


# speedup vs baseline: 2.7518x; 1.0051x over previous
import functools

import jax
import jax.numpy as jnp
from jax.experimental import pallas as pl
from jax.experimental.pallas import tpu as pltpu

_NUM_NEIGHBORS = 9
_KB1 = 8000
_KB2 = 20000
_BIG = 2 ** 30


def _stage1_kernel(xm2_ref, xn_ref, y_ref, yn_ref, iota_ref,
                   s_ref, idx_ref, sc_ref, *, n_blocks):
    k = pl.program_id(0)
    prod = jax.lax.dot_general(
        y_ref[...], xm2_ref[...], (((1,), (1,)), ((), ())),
        preferred_element_type=jnp.float32,
    )
    res = (xn_ref[...] + prod) + yn_ref[...]
    bmin = jnp.min(res, axis=0)
    base = jnp.maximum(bmin, 0.0)
    s = jnp.sqrt(base)
    bb = jax.lax.bitcast_convert_type(base, jnp.int32)
    hi = base
    for d in range(1, 7):
        hc = jax.lax.bitcast_convert_type(bb + d, jnp.float32)
        hi = jnp.where(jnp.sqrt(hc) == s, hc, hi)
    ii = jnp.where(res <= hi[None, :], iota_ref[...], _BIG)
    jloc = jnp.min(ii, axis=0) + k * _KB1

    @pl.when(k == 0)
    def _():
        s_ref[...] = s[None, :]
        idx_ref[...] = jloc[None, :]

    @pl.when(k > 0)
    def _():
        cur = s_ref[0, :]
        better = s < cur
        s_ref[...] = jnp.where(better, s, cur)[None, :]
        idx_ref[...] = jnp.where(better, jloc, idx_ref[0, :])[None, :]

    @pl.when(k == n_blocks - 1)
    def _():
        ps = s_ref[0, :]
        loc = idx_ref[0, :]
        i_n = jax.lax.broadcasted_iota(jnp.int32, ps.shape, 0)
        mp = jnp.min(jnp.where(ps == jnp.max(ps), i_n, _BIG))
        nn = jnp.sum(jnp.where(i_n == mp, loc, 0))
        sc_ref[...] = jnp.stack([mp, nn])[None, :]


def _stage2_kernel(s_ref, emb_ref, nnrow_ref, y_ref, yn_ref, ps_ref,
                   pred_ref, d2_scr, dr_scr, *, n_blocks):
    k = pl.program_id(0)
    q = jnp.concatenate([nnrow_ref[0], emb_ref[0]], axis=0)
    qn = jnp.sum(q * q, axis=1, keepdims=True)
    prod = jax.lax.dot_general(
        -2.0 * q, y_ref[...], (((1,), (1,)), ((), ())),
        preferred_element_type=jnp.float32,
    )
    res = (qn + prod) + yn_ref[0]
    dist = jnp.sqrt(jnp.maximum(res, 0.0))
    d2_scr[k, :] = dist[0, :]
    dr_scr[k, :] = dist[1, :]

    @pl.when(k == n_blocks - 1)
    def _():
        d2 = d2_scr[...]
        dr = dr_scr[...]
        fi = (jax.lax.broadcasted_iota(jnp.int32, d2.shape, 0) * _KB2
              + jax.lax.broadcasted_iota(jnp.int32, d2.shape, 1))
        d3 = []
        for _ in range(_NUM_NEIGHBORS):
            m = jnp.min(d2)
            fmin = jnp.min(jnp.where(d2 == m, fi, _BIG))
            sel = fi == fmin
            d3.append(jnp.sum(jnp.where(sel, dr, 0.0)))
            d2 = jnp.where(sel, jnp.inf, d2)
        mx = d3[0]
        for v in d3[1:]:
            mx = jnp.maximum(mx, v)
        denom = jnp.float32(0.0)
        for v in d3:
            denom = denom + jnp.exp(v - mx)
        w = 1.0 - jnp.exp(d3[0] - mx) / denom
        score = jnp.max(ps_ref[...])
        pred_ref[...] = jnp.reshape(w * score, (1, 1))


def kernel(embedding, memory_bank):
    n, d = embedding.shape
    k_total = memory_bank.shape[0]
    g1 = k_total // _KB1
    g2 = k_total // _KB2

    xn = jnp.sum(embedding ** 2, axis=-1, keepdims=True)
    yn = jnp.sum(memory_bank ** 2, axis=-1)
    xm2 = -2.0 * embedding
    xn_row = xn.reshape(1, n)
    yn_col = yn.reshape(k_total, 1)
    yn2 = yn.reshape(g2, 1, _KB2)
    iota1 = jax.lax.iota(jnp.int32, _KB1).reshape(_KB1, 1)

    ps, loc, sc = pl.pallas_call(
        functools.partial(_stage1_kernel, n_blocks=g1),
        grid=(g1,),
        in_specs=[
            pl.BlockSpec((n, d), lambda k: (0, 0)),
            pl.BlockSpec((1, n), lambda k: (0, 0)),
            pl.BlockSpec((_KB1, d), lambda k: (k, 0)),
            pl.BlockSpec((_KB1, 1), lambda k: (k, 0)),
            pl.BlockSpec((_KB1, 1), lambda k: (0, 0)),
        ],
        out_specs=[
            pl.BlockSpec((1, n), lambda k: (0, 0)),
            pl.BlockSpec((1, n), lambda k: (0, 0)),
            pl.BlockSpec((1, 2), lambda k: (0, 0)),
        ],
        out_shape=[
            jax.ShapeDtypeStruct((1, n), jnp.float32),
            jax.ShapeDtypeStruct((1, n), jnp.int32),
            jax.ShapeDtypeStruct((1, 2), jnp.int32),
        ],
    )(xm2, xn_row, memory_bank, yn_col, iota1)

    pred = pl.pallas_call(
        functools.partial(_stage2_kernel, n_blocks=g2),
        grid_spec=pltpu.PrefetchScalarGridSpec(
            num_scalar_prefetch=1,
            grid=(g2,),
            in_specs=[
                pl.BlockSpec((1, 1, d), lambda k, s: (s[0], 0, 0)),
                pl.BlockSpec((1, 1, d), lambda k, s: (s[1], 0, 0)),
                pl.BlockSpec((_KB2, d), lambda k, s: (k, 0)),
                pl.BlockSpec((1, 1, _KB2), lambda k, s: (k, 0, 0)),
                pl.BlockSpec((1, n), lambda k, s: (0, 0)),
            ],
            out_specs=pl.BlockSpec((1, 1), lambda k, s: (0, 0)),
            scratch_shapes=[
                pltpu.VMEM((g2, _KB2), jnp.float32),
                pltpu.VMEM((g2, _KB2), jnp.float32),
            ],
        ),
        out_shape=jax.ShapeDtypeStruct((1, 1), jnp.float32),
    )(sc.reshape(2), embedding.reshape(n, 1, d),
      memory_bank.reshape(k_total, 1, d), memory_bank, yn2, ps)

    pred_score = pred.reshape(1)
    anomaly_map = ps.reshape(1, 1, 28, 28)
    return pred_score, anomaly_map, loc
